# Initial kernel scaffold; baseline (speedup 1.0000x reference)
import jax
import jax.numpy as jnp
from jax.experimental import pallas as pl


def kernel(x, emb, wg, w1, b1, w2, b2, fcw, fcb):
    raise NotImplementedError("write your pallas kernel here")


# speedup vs baseline: 12.6177x; 12.6177x over previous
import functools

import jax
import jax.numpy as jnp
from jax import lax
from jax.experimental import pallas as pl
from jax.experimental.pallas import tpu as pltpu
from jax.experimental.pallas import tpu_sc as plsc

_V = 100000
_D = 128
_E = 8
_FFN = 512
_OUT = 2
_B = 4096
_L = 200

_NC, _NS = 2, 16
_NW = _NC * _NS
_TPW = _B // _NW
_CH = _L // 2


def _pool_sc(x2, emb):
    mesh = plsc.VectorSubcoreMesh(core_axis_name="c", subcore_axis_name="s")

    @functools.partial(
        pl.kernel,
        out_type=jax.ShapeDtypeStruct((_B, _D), jnp.float32),
        mesh=mesh,
        scratch_types=[
            pltpu.VMEM((2 * _TPW, _CH), jnp.int32),
            pltpu.VMEM((_CH, _D), jnp.float32),
            pltpu.VMEM((_CH, _D), jnp.float32),
            pltpu.VMEM((_CH, _D), jnp.float32),
            pltpu.VMEM((_CH, _D), jnp.float32),
            pltpu.VMEM((_TPW, _D), jnp.float32),
            pltpu.SemaphoreType.DMA,
            pltpu.SemaphoreType.DMA,
        ],
    )
    def k(x2_hbm, emb_hbm, out_hbm, idx_v, r0a, r0b, r1a, r1b, out_v, sem0, sem1):
        wid = lax.axis_index("s") * _NC + lax.axis_index("c")
        pltpu.sync_copy(x2_hbm.at[pl.ds(wid * (2 * _TPW), 2 * _TPW)], idx_v)
        pltpu.async_copy(emb_hbm.at[idx_v.at[0]], r0a, sem0)
        pltpu.async_copy(emb_hbm.at[idx_v.at[1]], r0b, sem0)
        pltpu.async_copy(emb_hbm.at[idx_v.at[2]], r1a, sem1)
        pltpu.async_copy(emb_hbm.at[idx_v.at[3]], r1b, sem1)

        @pl.loop(0, _TPW, step=2)
        def _(t0):
            for dt, ra, rb, sem in ((0, r0a, r0b, sem0), (1, r1a, r1b, sem1)):
                t = t0 + dt
                pltpu.make_async_copy(emb_hbm.at[idx_v.at[0]], ra, sem).wait()
                pltpu.make_async_copy(emb_hbm.at[idx_v.at[0]], rb, sem).wait()

                def body(l, accs):
                    return tuple(
                        accs[d] + ra[l, pl.ds(16 * d, 16)] + rb[l, pl.ds(16 * d, 16)]
                        for d in range(8)
                    )

                accs = lax.fori_loop(
                    0, _CH, body,
                    tuple(jnp.zeros((16,), jnp.float32) for _ in range(8)),
                )
                for d in range(8):
                    out_v[t, pl.ds(16 * d, 16)] = accs[d] * (1.0 / _L)

                @pl.when(t + 2 < _TPW)
                def _fire():
                    pltpu.async_copy(emb_hbm.at[idx_v.at[2 * (t + 2)]], ra, sem)
                    pltpu.async_copy(emb_hbm.at[idx_v.at[2 * (t + 2) + 1]], rb, sem)

        pltpu.sync_copy(out_v, out_hbm.at[pl.ds(wid * _TPW, _TPW)])

    return k(x2, emb)


def _moe_tc(pooled, wg, w1, b1, w2, b2, fcw, fcb2):
    BT = 512

    def kfn(p_ref, wg_ref, w1_ref, b1_ref, w2_ref, b2_ref, fcw_ref, fcb_ref, o_ref):
        xb = p_ref[...]
        logits = jnp.dot(xb, wg_ref[...], preferred_element_type=jnp.float32)
        m = jnp.max(logits, axis=-1, keepdims=True)
        gate = 1.0 / jnp.sum(jnp.exp(logits - m), axis=-1, keepdims=True)
        iot = lax.broadcasted_iota(jnp.int32, logits.shape, 1)
        sel = jnp.min(jnp.where(logits >= m, iot, _E), axis=-1, keepdims=True)
        acc = jnp.zeros((BT, _D), jnp.float32)
        for e in range(_E):
            b1r = b1_ref[pl.ds(e, 1), :]
            b2r = b2_ref[pl.ds(e, 1), :]
            h = jnp.dot(xb, w1_ref[e], preferred_element_type=jnp.float32) + b1r
            h = jnp.maximum(h, 0.0)
            o = jnp.dot(h, w2_ref[e], preferred_element_type=jnp.float32) + b2r
            acc = acc + o * (sel == e).astype(jnp.float32)
        moe = acc * gate
        out = jnp.dot(moe, fcw_ref[...], preferred_element_type=jnp.float32)
        out = out + fcb_ref[...]
        mm = jnp.max(out, axis=-1, keepdims=True)
        out = out - mm
        o_ref[...] = out - jnp.log(jnp.sum(jnp.exp(out), axis=-1, keepdims=True))

    return pl.pallas_call(
        kfn,
        grid=(_B // BT,),
        in_specs=[
            pl.BlockSpec((BT, _D), lambda i: (i, 0)),
            pl.BlockSpec((_D, _E), lambda i: (0, 0)),
            pl.BlockSpec((_E, _D, _FFN), lambda i: (0, 0, 0)),
            pl.BlockSpec((_E, _FFN), lambda i: (0, 0)),
            pl.BlockSpec((_E, _FFN, _D), lambda i: (0, 0, 0)),
            pl.BlockSpec((_E, _D), lambda i: (0, 0)),
            pl.BlockSpec((_D, _OUT), lambda i: (0, 0)),
            pl.BlockSpec((1, _OUT), lambda i: (0, 0)),
        ],
        out_specs=pl.BlockSpec((BT, _OUT), lambda i: (i, 0)),
        out_shape=jax.ShapeDtypeStruct((_B, _OUT), jnp.float32),
    )(pooled, wg, w1, b1, w2, b2, fcw, fcb2)


def kernel(x, emb, wg, w1, b1, w2, b2, fcw, fcb):
    x2 = x.astype(jnp.int32).reshape(2 * _B, _CH)
    pooled = _pool_sc(x2, emb)
    return _moe_tc(pooled, wg, w1, b1, w2, b2, fcw, fcb.reshape(1, _OUT))
